# baseline (speedup 1.0000x reference)
import functools

import jax
import jax.numpy as jnp
from jax import lax
from jax.experimental import pallas as pl
from jax.experimental.pallas import tpu as pltpu
from jax.experimental.pallas import tpu_sc as plsc

R = 1024
N = 32768
L = 16
NW = 32
R_SC = 0
ROWS_PER_W = R_SC // NW
GROUPS = (ROWS_PER_W + L - 1) // L
UNROLL = 8
ITERS = N // (L * UNROLL)

G_TC = (R - R_SC) // 8
TCU = 8
CH = N // 128
BIG = 2**30


def _vperm(x, perm):
    return lax.gather(
        x,
        perm[:, None],
        dimension_numbers=lax.GatherDimensionNumbers(
            offset_dims=(), collapsed_slice_dims=(0,), start_index_map=(0,)
        ),
        slice_sizes=(1,),
        mode=lax.GatherScatterMode.PROMISE_IN_BOUNDS,
    )


def _merge(am, ai, bm, bi):
    take = (bm > am) | ((bm == am) & (bi < ai))
    return jnp.where(take, bm, am), jnp.where(take, bi, ai)


def _row_argmax(buf, iota):
    neg_inf = jnp.full((L,), -jnp.inf, dtype=jnp.float32)
    zero_i = jnp.zeros((L,), dtype=jnp.int32)

    def step(t, carry):
        viter = carry[-1]
        cms = list(carry[0])
        cis = list(carry[1])
        for j in range(UNROLL):
            v = buf[pl.ds(t * (L * UNROLL) + j * L, L)]
            gt = v > cms[j]
            cms[j] = jnp.where(gt, v, cms[j])
            cis[j] = jnp.where(gt, viter, cis[j])
        return (tuple(cms), tuple(cis), viter + 1)

    init = (tuple([neg_inf] * UNROLL), tuple([zero_i] * UNROLL), zero_i)
    cms, cis, _ = lax.fori_loop(0, ITERS, step, init)

    bm, bi = None, None
    for j in range(UNROLL):
        idx = (cis[j] << 7) | (j << 4) | iota
        if bm is None:
            bm, bi = cms[j], idx
        else:
            bm, bi = _merge(bm, bi, cms[j], idx)

    for off in (8, 4, 2, 1):
        perm = iota ^ off
        bm, bi = _merge(bm, bi, _vperm(bm, perm), _vperm(bi, perm))
    return bi


def _sc_argmax(x_hbm, out_hbm, buf_a, buf_b, res_v, sem_a, sem_b):
    c = lax.axis_index("c")
    s = lax.axis_index("s")
    wid = s * 2 + c
    base = wid * ROWS_PER_W
    iota = lax.iota(jnp.int32, L)

    bufs = (buf_a, buf_b)
    sems = (sem_a, sem_b)

    def start(r, b):
        return pltpu.async_copy(x_hbm.at[base + r], bufs[b], sems[b])

    handles = [start(0, 0), None]
    res = [jnp.zeros((L,), jnp.int32)] * GROUPS
    for r in range(ROWS_PER_W):
        b = r & 1
        if r + 1 < ROWS_PER_W:
            handles[1 - b] = start(r + 1, 1 - b)
        handles[b].wait()
        p = _row_argmax(bufs[b], iota)
        res[r // L] = jnp.where(iota == (r % L), p, res[r // L])

    for g in range(GROUPS):
        res_v[pl.ds(g * L, L)] = res[g]
    pltpu.sync_copy(res_v, out_hbm.at[wid])


def _tc_body(x_ref, o_ref):
    lane = lax.broadcasted_iota(jnp.int32, (8, 128), 1)
    neg_inf = jnp.full((8, 128), -jnp.inf, jnp.float32)

    def step(t, carry):
        vals = list(carry[0])
        idxs = list(carry[1])
        for j in range(TCU):
            c = t * TCU + j
            v = x_ref[:, pl.ds(c * 128, 128)]
            gt = v > vals[j]
            vals[j] = jnp.where(gt, v, vals[j])
            idxs[j] = jnp.where(gt, c * 128 + lane, idxs[j])
        return (tuple(vals), tuple(idxs))

    init = (
        tuple([neg_inf] * TCU),
        tuple([jnp.zeros((8, 128), jnp.int32)] * TCU),
    )
    vals, idxs = lax.fori_loop(0, CH // TCU, step, init)

    bm, bi = vals[0], idxs[0]
    for j in range(1, TCU):
        bm, bi = _merge(bm, bi, vals[j], idxs[j])

    m = jnp.max(bm, axis=1, keepdims=True)
    best = jnp.min(jnp.where(bm == m, bi, BIG), axis=1)
    o_ref[...] = best.reshape(1, 1, 8)


@jax.jit
def _argmax_split(x2d):
    if R_SC == 0:
        tc_only = pl.pallas_call(
            _tc_body,
            grid=(G_TC,),
            in_specs=[pl.BlockSpec((8, N), lambda i: (i, 0))],
            out_specs=pl.BlockSpec((1, 1, 8), lambda i: (i, 0, 0)),
            out_shape=jax.ShapeDtypeStruct((G_TC, 1, 8), jnp.int32),
        )
        return tc_only(x2d).reshape(R)

    mesh = plsc.VectorSubcoreMesh(core_axis_name="c", subcore_axis_name="s")
    sc_f = pl.kernel(
        _sc_argmax,
        out_type=jax.ShapeDtypeStruct((NW, GROUPS * L), jnp.int32),
        mesh=mesh,
        scratch_types=[
            pltpu.VMEM((N,), jnp.float32),
            pltpu.VMEM((N,), jnp.float32),
            pltpu.VMEM((GROUPS * L,), jnp.int32),
            pltpu.SemaphoreType.DMA,
            pltpu.SemaphoreType.DMA,
        ],
    )
    idx_sc = sc_f(x2d)[:, :ROWS_PER_W].reshape(R_SC)

    tc_f = pl.pallas_call(
        _tc_body,
        grid=(G_TC,),
        in_specs=[
            pl.BlockSpec((8, N), lambda i: (R_SC // 8 + i, 0)),
        ],
        out_specs=pl.BlockSpec((1, 1, 8), lambda i: (i, 0, 0)),
        out_shape=jax.ShapeDtypeStruct((G_TC, 1, 8), jnp.int32),
    )
    idx_tc = tc_f(x2d).reshape(R - R_SC)
    return jnp.concatenate([idx_sc, idx_tc])


def kernel(x):
    idx = _argmax_split(x.reshape(R, N))
    return idx.reshape(128, 8).astype(jnp.int64)


# speedup vs baseline: 1.3339x; 1.3339x over previous
import functools

import jax
import jax.numpy as jnp
from jax import lax
from jax.experimental import pallas as pl
from jax.experimental.pallas import tpu as pltpu
from jax.experimental.pallas import tpu_sc as plsc

R = 1024
N = 32768
L = 16
NW = 32
R_SC = 0
ROWS_PER_W = R_SC // NW
GROUPS = (ROWS_PER_W + L - 1) // L
UNROLL = 8
ITERS = N // (L * UNROLL)

G_TC = (R - R_SC) // 8
TCU = 8
CH = N // 128
BIG = 2**30


def _vperm(x, perm):
    return lax.gather(
        x,
        perm[:, None],
        dimension_numbers=lax.GatherDimensionNumbers(
            offset_dims=(), collapsed_slice_dims=(0,), start_index_map=(0,)
        ),
        slice_sizes=(1,),
        mode=lax.GatherScatterMode.PROMISE_IN_BOUNDS,
    )


def _merge(am, ai, bm, bi):
    take = (bm > am) | ((bm == am) & (bi < ai))
    return jnp.where(take, bm, am), jnp.where(take, bi, ai)


def _row_argmax(buf, iota):
    neg_inf = jnp.full((L,), -jnp.inf, dtype=jnp.float32)
    zero_i = jnp.zeros((L,), dtype=jnp.int32)

    def step(t, carry):
        viter = carry[-1]
        cms = list(carry[0])
        cis = list(carry[1])
        for j in range(UNROLL):
            v = buf[pl.ds(t * (L * UNROLL) + j * L, L)]
            gt = v > cms[j]
            cms[j] = jnp.where(gt, v, cms[j])
            cis[j] = jnp.where(gt, viter, cis[j])
        return (tuple(cms), tuple(cis), viter + 1)

    init = (tuple([neg_inf] * UNROLL), tuple([zero_i] * UNROLL), zero_i)
    cms, cis, _ = lax.fori_loop(0, ITERS, step, init)

    bm, bi = None, None
    for j in range(UNROLL):
        idx = (cis[j] << 7) | (j << 4) | iota
        if bm is None:
            bm, bi = cms[j], idx
        else:
            bm, bi = _merge(bm, bi, cms[j], idx)

    for off in (8, 4, 2, 1):
        perm = iota ^ off
        bm, bi = _merge(bm, bi, _vperm(bm, perm), _vperm(bi, perm))
    return bi


def _sc_argmax(x_hbm, out_hbm, buf_a, buf_b, res_v, sem_a, sem_b):
    c = lax.axis_index("c")
    s = lax.axis_index("s")
    wid = s * 2 + c
    base = wid * ROWS_PER_W
    iota = lax.iota(jnp.int32, L)

    bufs = (buf_a, buf_b)
    sems = (sem_a, sem_b)

    def start(r, b):
        return pltpu.async_copy(x_hbm.at[base + r], bufs[b], sems[b])

    handles = [start(0, 0), None]
    res = [jnp.zeros((L,), jnp.int32)] * GROUPS
    for r in range(ROWS_PER_W):
        b = r & 1
        if r + 1 < ROWS_PER_W:
            handles[1 - b] = start(r + 1, 1 - b)
        handles[b].wait()
        p = _row_argmax(bufs[b], iota)
        res[r // L] = jnp.where(iota == (r % L), p, res[r // L])

    for g in range(GROUPS):
        res_v[pl.ds(g * L, L)] = res[g]
    pltpu.sync_copy(res_v, out_hbm.at[wid])


def _tc_body(x_ref, o_ref):
    lane = lax.broadcasted_iota(jnp.int32, (8, 128), 1)
    neg_inf = jnp.full((8, 128), -jnp.inf, jnp.float32)

    def step(t, carry):
        vals = list(carry[0])
        idxs = list(carry[1])
        for j in range(TCU):
            c = t * TCU + j
            v = x_ref[:, pl.ds(c * 128, 128)]
            gt = v > vals[j]
            vals[j] = jnp.where(gt, v, vals[j])
            idxs[j] = jnp.where(gt, c * 128 + lane, idxs[j])
        return (tuple(vals), tuple(idxs))

    init = (
        tuple([neg_inf] * TCU),
        tuple([jnp.zeros((8, 128), jnp.int32)] * TCU),
    )
    vals, idxs = lax.fori_loop(0, CH // TCU, step, init)

    bm, bi = vals[0], idxs[0]
    for j in range(1, TCU):
        bm, bi = _merge(bm, bi, vals[j], idxs[j])

    m = jnp.max(bm, axis=1, keepdims=True)
    best = jnp.min(jnp.where(bm == m, bi, BIG), axis=1)
    o_ref[...] = best.reshape(1, 1, 8)


def _tc_group_argmax(x_ref, row0):
    lane = lax.broadcasted_iota(jnp.int32, (8, 128), 1)
    neg_inf = jnp.full((8, 128), -jnp.inf, jnp.float32)

    def step(t, carry):
        vals = list(carry[0])
        idxs = list(carry[1])
        for j in range(TCU):
            c = t * TCU + j
            v = x_ref[pl.ds(row0, 8), pl.ds(c * 128, 128)]
            gt = v > vals[j]
            vals[j] = jnp.where(gt, v, vals[j])
            idxs[j] = jnp.where(gt, c * 128 + lane, idxs[j])
        return (tuple(vals), tuple(idxs))

    init = (
        tuple([neg_inf] * TCU),
        tuple([jnp.zeros((8, 128), jnp.int32)] * TCU),
    )
    vals, idxs = lax.fori_loop(0, CH // TCU, step, init)

    bm, bi = vals[0], idxs[0]
    for j in range(1, TCU):
        bm, bi = _merge(bm, bi, vals[j], idxs[j])

    m = jnp.max(bm, axis=1, keepdims=True)
    return jnp.min(jnp.where(bm == m, bi, BIG), axis=1)


def _tc_body32(x_ref, o_ref):
    outs = [_tc_group_argmax(x_ref, r * 8) for r in range(4)]
    o_ref[...] = jnp.concatenate(outs).reshape(1, 1, 32)


@jax.jit
def _argmax_split(x2d):
    if R_SC == 0:
        g32 = R // 32
        tc_only = pl.pallas_call(
            _tc_body32,
            grid=(g32,),
            in_specs=[pl.BlockSpec((32, N), lambda i: (i, 0))],
            out_specs=pl.BlockSpec((1, 1, 32), lambda i: (i, 0, 0)),
            out_shape=jax.ShapeDtypeStruct((g32, 1, 32), jnp.int32),
        )
        return tc_only(x2d).reshape(R)

    mesh = plsc.VectorSubcoreMesh(core_axis_name="c", subcore_axis_name="s")
    sc_f = pl.kernel(
        _sc_argmax,
        out_type=jax.ShapeDtypeStruct((NW, GROUPS * L), jnp.int32),
        mesh=mesh,
        scratch_types=[
            pltpu.VMEM((N,), jnp.float32),
            pltpu.VMEM((N,), jnp.float32),
            pltpu.VMEM((GROUPS * L,), jnp.int32),
            pltpu.SemaphoreType.DMA,
            pltpu.SemaphoreType.DMA,
        ],
    )
    idx_sc = sc_f(x2d)[:, :ROWS_PER_W].reshape(R_SC)

    tc_f = pl.pallas_call(
        _tc_body,
        grid=(G_TC,),
        in_specs=[
            pl.BlockSpec((8, N), lambda i: (R_SC // 8 + i, 0)),
        ],
        out_specs=pl.BlockSpec((1, 1, 8), lambda i: (i, 0, 0)),
        out_shape=jax.ShapeDtypeStruct((G_TC, 1, 8), jnp.int32),
    )
    idx_tc = tc_f(x2d).reshape(R - R_SC)
    return jnp.concatenate([idx_sc, idx_tc])


def kernel(x):
    idx = _argmax_split(x.reshape(R, N))
    return idx.reshape(128, 8).astype(jnp.int64)


# speedup vs baseline: 1.9340x; 1.4498x over previous
import functools

import jax
import jax.numpy as jnp
from jax import lax
from jax.experimental import pallas as pl
from jax.experimental.pallas import tpu as pltpu
from jax.experimental.pallas import tpu_sc as plsc

R = 1024
N = 32768
L = 16
NW = 32
R_SC = 576
ROWS_PER_W = R_SC // NW
GROUPS = (ROWS_PER_W + L - 1) // L
UNROLL = 8
ITERS = N // (L * UNROLL)

G_TC = (R - R_SC) // 8
TCU = 8
CH = N // 128
BIG = 2**30


def _vperm(x, perm):
    return lax.gather(
        x,
        perm[:, None],
        dimension_numbers=lax.GatherDimensionNumbers(
            offset_dims=(), collapsed_slice_dims=(0,), start_index_map=(0,)
        ),
        slice_sizes=(1,),
        mode=lax.GatherScatterMode.PROMISE_IN_BOUNDS,
    )


def _merge(am, ai, bm, bi):
    take = (bm > am) | ((bm == am) & (bi < ai))
    return jnp.where(take, bm, am), jnp.where(take, bi, ai)


def _row_argmax(buf, iota):
    neg_inf = jnp.full((L,), -jnp.inf, dtype=jnp.float32)
    zero_i = jnp.zeros((L,), dtype=jnp.int32)

    def step(t, carry):
        viter = carry[-1]
        cms = list(carry[0])
        cis = list(carry[1])
        for j in range(UNROLL):
            v = buf[pl.ds(t * (L * UNROLL) + j * L, L)]
            gt = v > cms[j]
            cms[j] = jnp.where(gt, v, cms[j])
            cis[j] = jnp.where(gt, viter, cis[j])
        return (tuple(cms), tuple(cis), viter + 1)

    init = (tuple([neg_inf] * UNROLL), tuple([zero_i] * UNROLL), zero_i)
    cms, cis, _ = lax.fori_loop(0, ITERS, step, init)

    bm, bi = None, None
    for j in range(UNROLL):
        idx = (cis[j] << 7) | (j << 4) | iota
        if bm is None:
            bm, bi = cms[j], idx
        else:
            bm, bi = _merge(bm, bi, cms[j], idx)

    for off in (8, 4, 2, 1):
        perm = iota ^ off
        bm, bi = _merge(bm, bi, _vperm(bm, perm), _vperm(bi, perm))
    return bi


def _sc_argmax(x_hbm, out_hbm, buf_a, buf_b, res_v, sem_a, sem_b):
    c = lax.axis_index("c")
    s = lax.axis_index("s")
    wid = s * 2 + c
    base = wid * ROWS_PER_W
    iota = lax.iota(jnp.int32, L)

    bufs = (buf_a, buf_b)
    sems = (sem_a, sem_b)

    def start(r, b):
        return pltpu.async_copy(x_hbm.at[base + r], bufs[b], sems[b])

    handles = [start(0, 0), None]
    res = [jnp.zeros((L,), jnp.int32)] * GROUPS
    for r in range(ROWS_PER_W):
        b = r & 1
        if r + 1 < ROWS_PER_W:
            handles[1 - b] = start(r + 1, 1 - b)
        handles[b].wait()
        p = _row_argmax(bufs[b], iota)
        res[r // L] = jnp.where(iota == (r % L), p, res[r // L])

    for g in range(GROUPS):
        res_v[pl.ds(g * L, L)] = res[g]
    pltpu.sync_copy(res_v, out_hbm.at[wid])


def _tc_body(x_ref, o_ref):
    lane = lax.broadcasted_iota(jnp.int32, (8, 128), 1)
    neg_inf = jnp.full((8, 128), -jnp.inf, jnp.float32)

    def step(t, carry):
        vals = list(carry[0])
        idxs = list(carry[1])
        for j in range(TCU):
            c = t * TCU + j
            v = x_ref[:, pl.ds(c * 128, 128)]
            gt = v > vals[j]
            vals[j] = jnp.where(gt, v, vals[j])
            idxs[j] = jnp.where(gt, c * 128 + lane, idxs[j])
        return (tuple(vals), tuple(idxs))

    init = (
        tuple([neg_inf] * TCU),
        tuple([jnp.zeros((8, 128), jnp.int32)] * TCU),
    )
    vals, idxs = lax.fori_loop(0, CH // TCU, step, init)

    bm, bi = vals[0], idxs[0]
    for j in range(1, TCU):
        bm, bi = _merge(bm, bi, vals[j], idxs[j])

    m = jnp.max(bm, axis=1, keepdims=True)
    best = jnp.min(jnp.where(bm == m, bi, BIG), axis=1)
    o_ref[...] = best.reshape(1, 1, 8)


def _tc_group_argmax(x_ref, row0):
    lane = lax.broadcasted_iota(jnp.int32, (8, 128), 1)
    neg_inf = jnp.full((8, 128), -jnp.inf, jnp.float32)

    def step(t, carry):
        vals = list(carry[0])
        idxs = list(carry[1])
        for j in range(TCU):
            c = t * TCU + j
            v = x_ref[pl.ds(row0, 8), pl.ds(c * 128, 128)]
            gt = v > vals[j]
            vals[j] = jnp.where(gt, v, vals[j])
            idxs[j] = jnp.where(gt, c * 128 + lane, idxs[j])
        return (tuple(vals), tuple(idxs))

    init = (
        tuple([neg_inf] * TCU),
        tuple([jnp.zeros((8, 128), jnp.int32)] * TCU),
    )
    vals, idxs = lax.fori_loop(0, CH // TCU, step, init)

    bm, bi = vals[0], idxs[0]
    for j in range(1, TCU):
        bm, bi = _merge(bm, bi, vals[j], idxs[j])

    m = jnp.max(bm, axis=1, keepdims=True)
    return jnp.min(jnp.where(bm == m, bi, BIG), axis=1)


def _tc_body32(x_ref, o_ref):
    outs = [_tc_group_argmax(x_ref, r * 8) for r in range(4)]
    o_ref[...] = jnp.concatenate(outs).reshape(1, 1, 32)


@jax.jit
def _argmax_split(x2d):
    if R_SC == 0:
        g32 = R // 32
        tc_only = pl.pallas_call(
            _tc_body32,
            grid=(g32,),
            in_specs=[pl.BlockSpec((32, N), lambda i: (i, 0))],
            out_specs=pl.BlockSpec((1, 1, 32), lambda i: (i, 0, 0)),
            out_shape=jax.ShapeDtypeStruct((g32, 1, 32), jnp.int32),
        )
        return tc_only(x2d).reshape(R)

    mesh = plsc.VectorSubcoreMesh(core_axis_name="c", subcore_axis_name="s")
    sc_f = pl.kernel(
        _sc_argmax,
        out_type=jax.ShapeDtypeStruct((NW, GROUPS * L), jnp.int32),
        mesh=mesh,
        scratch_types=[
            pltpu.VMEM((N,), jnp.float32),
            pltpu.VMEM((N,), jnp.float32),
            pltpu.VMEM((GROUPS * L,), jnp.int32),
            pltpu.SemaphoreType.DMA,
            pltpu.SemaphoreType.DMA,
        ],
    )
    idx_sc = sc_f(x2d)[:, :ROWS_PER_W].reshape(R_SC)

    g32 = (R - R_SC) // 32
    tc_f = pl.pallas_call(
        _tc_body32,
        grid=(g32,),
        in_specs=[
            pl.BlockSpec((32, N), lambda i: (R_SC // 32 + i, 0)),
        ],
        out_specs=pl.BlockSpec((1, 1, 32), lambda i: (i, 0, 0)),
        out_shape=jax.ShapeDtypeStruct((g32, 1, 32), jnp.int32),
    )
    idx_tc = tc_f(x2d).reshape(R - R_SC)
    return jnp.concatenate([idx_sc, idx_tc])


def kernel(x):
    idx = _argmax_split(x.reshape(R, N))
    return idx.reshape(128, 8).astype(jnp.int64)


# speedup vs baseline: 1.9619x; 1.0144x over previous
import functools

import jax
import jax.numpy as jnp
from jax import lax
from jax.experimental import pallas as pl
from jax.experimental.pallas import tpu as pltpu
from jax.experimental.pallas import tpu_sc as plsc

R = 1024
N = 32768
L = 16
NW = 32
R_SC = 512
ROWS_PER_W = R_SC // NW
GROUPS = (ROWS_PER_W + L - 1) // L
UNROLL = 8
ITERS = N // (L * UNROLL)

G_TC = (R - R_SC) // 8
TCU = 8
CH = N // 128
BIG = 2**30


def _vperm(x, perm):
    return lax.gather(
        x,
        perm[:, None],
        dimension_numbers=lax.GatherDimensionNumbers(
            offset_dims=(), collapsed_slice_dims=(0,), start_index_map=(0,)
        ),
        slice_sizes=(1,),
        mode=lax.GatherScatterMode.PROMISE_IN_BOUNDS,
    )


def _merge(am, ai, bm, bi):
    take = (bm > am) | ((bm == am) & (bi < ai))
    return jnp.where(take, bm, am), jnp.where(take, bi, ai)


def _row_argmax(buf, iota):
    neg_inf = jnp.full((L,), -jnp.inf, dtype=jnp.float32)
    zero_i = jnp.zeros((L,), dtype=jnp.int32)

    def step(t, carry):
        viter = carry[-1]
        cms = list(carry[0])
        cis = list(carry[1])
        for j in range(UNROLL):
            v = buf[pl.ds(t * (L * UNROLL) + j * L, L)]
            gt = v > cms[j]
            cms[j] = jnp.where(gt, v, cms[j])
            cis[j] = jnp.where(gt, viter, cis[j])
        return (tuple(cms), tuple(cis), viter + 1)

    init = (tuple([neg_inf] * UNROLL), tuple([zero_i] * UNROLL), zero_i)
    cms, cis, _ = lax.fori_loop(0, ITERS, step, init)

    bm, bi = None, None
    for j in range(UNROLL):
        idx = (cis[j] << 7) | (j << 4) | iota
        if bm is None:
            bm, bi = cms[j], idx
        else:
            bm, bi = _merge(bm, bi, cms[j], idx)

    for off in (8, 4, 2, 1):
        perm = iota ^ off
        bm, bi = _merge(bm, bi, _vperm(bm, perm), _vperm(bi, perm))
    return bi


def _sc_argmax(x_hbm, out_hbm, buf_a, buf_b, res_v, sem_a, sem_b):
    c = lax.axis_index("c")
    s = lax.axis_index("s")
    wid = s * 2 + c
    base = wid * ROWS_PER_W
    iota = lax.iota(jnp.int32, L)

    bufs = (buf_a, buf_b)
    sems = (sem_a, sem_b)

    def start(r, b):
        return pltpu.async_copy(x_hbm.at[base + r], bufs[b], sems[b])

    handles = [start(0, 0), None]
    res = [jnp.zeros((L,), jnp.int32)] * GROUPS
    for r in range(ROWS_PER_W):
        b = r & 1
        if r + 1 < ROWS_PER_W:
            handles[1 - b] = start(r + 1, 1 - b)
        handles[b].wait()
        p = _row_argmax(bufs[b], iota)
        res[r // L] = jnp.where(iota == (r % L), p, res[r // L])

    for g in range(GROUPS):
        res_v[pl.ds(g * L, L)] = res[g]
    pltpu.sync_copy(res_v, out_hbm.at[wid])


def _tc_body(x_ref, o_ref):
    lane = lax.broadcasted_iota(jnp.int32, (8, 128), 1)
    neg_inf = jnp.full((8, 128), -jnp.inf, jnp.float32)

    def step(t, carry):
        vals = list(carry[0])
        idxs = list(carry[1])
        for j in range(TCU):
            c = t * TCU + j
            v = x_ref[:, pl.ds(c * 128, 128)]
            gt = v > vals[j]
            vals[j] = jnp.where(gt, v, vals[j])
            idxs[j] = jnp.where(gt, c * 128 + lane, idxs[j])
        return (tuple(vals), tuple(idxs))

    init = (
        tuple([neg_inf] * TCU),
        tuple([jnp.zeros((8, 128), jnp.int32)] * TCU),
    )
    vals, idxs = lax.fori_loop(0, CH // TCU, step, init)

    bm, bi = vals[0], idxs[0]
    for j in range(1, TCU):
        bm, bi = _merge(bm, bi, vals[j], idxs[j])

    m = jnp.max(bm, axis=1, keepdims=True)
    best = jnp.min(jnp.where(bm == m, bi, BIG), axis=1)
    o_ref[...] = best.reshape(1, 1, 8)


def _tc_group_argmax(x_ref, row0):
    lane = lax.broadcasted_iota(jnp.int32, (8, 128), 1)
    neg_inf = jnp.full((8, 128), -jnp.inf, jnp.float32)

    def step(t, carry):
        vals = list(carry[0])
        idxs = list(carry[1])
        for j in range(TCU):
            c = t * TCU + j
            v = x_ref[pl.ds(row0, 8), pl.ds(c * 128, 128)]
            gt = v > vals[j]
            vals[j] = jnp.where(gt, v, vals[j])
            idxs[j] = jnp.where(gt, c * 128 + lane, idxs[j])
        return (tuple(vals), tuple(idxs))

    init = (
        tuple([neg_inf] * TCU),
        tuple([jnp.zeros((8, 128), jnp.int32)] * TCU),
    )
    vals, idxs = lax.fori_loop(0, CH // TCU, step, init)

    bm, bi = vals[0], idxs[0]
    for j in range(1, TCU):
        bm, bi = _merge(bm, bi, vals[j], idxs[j])

    m = jnp.max(bm, axis=1, keepdims=True)
    return jnp.min(jnp.where(bm == m, bi, BIG), axis=1)


def _tc_body32(x_ref, o_ref):
    outs = [_tc_group_argmax(x_ref, r * 8) for r in range(4)]
    o_ref[...] = jnp.concatenate(outs).reshape(1, 1, 32)


BR = 16
NBUF = 4


def _tc_manual(x_ref, o_ref, b0, b1, b2, b3, s0, s1, s2, s3):
    bufs = (b0, b1, b2, b3)
    sems = (s0, s1, s2, s3)
    nstep = (R - R_SC) // BR

    def dma(t, j):
        return pltpu.make_async_copy(
            x_ref.at[pl.ds(R_SC + t * BR, BR), :], bufs[j], sems[j]
        )

    for j in range(NBUF):
        dma(j, j).start()

    def outer(k, carry):
        for j in range(NBUF):
            t = k * NBUF + j
            dma(t, j).wait()
            for sub in range(BR // 8):
                best = _tc_group_argmax(bufs[j], sub * 8)
                g = t * (BR // 8) + sub
                o_ref[pl.ds(g, 1)] = best.reshape(1, 1, 8)
            nt = t + NBUF

            @pl.when(nt < nstep)
            def _():
                dma(nt, j).start()

        return carry

    lax.fori_loop(0, nstep // NBUF, outer, 0)


@jax.jit
def _argmax_split(x2d):
    if R_SC == 0:
        g32 = R // 32
        tc_only = pl.pallas_call(
            _tc_body32,
            grid=(g32,),
            in_specs=[pl.BlockSpec((32, N), lambda i: (i, 0))],
            out_specs=pl.BlockSpec((1, 1, 32), lambda i: (i, 0, 0)),
            out_shape=jax.ShapeDtypeStruct((g32, 1, 32), jnp.int32),
        )
        return tc_only(x2d).reshape(R)

    mesh = plsc.VectorSubcoreMesh(core_axis_name="c", subcore_axis_name="s")
    sc_f = pl.kernel(
        _sc_argmax,
        out_type=jax.ShapeDtypeStruct((NW, GROUPS * L), jnp.int32),
        mesh=mesh,
        scratch_types=[
            pltpu.VMEM((N,), jnp.float32),
            pltpu.VMEM((N,), jnp.float32),
            pltpu.VMEM((GROUPS * L,), jnp.int32),
            pltpu.SemaphoreType.DMA,
            pltpu.SemaphoreType.DMA,
        ],
    )
    idx_sc = sc_f(x2d)[:, :ROWS_PER_W].reshape(R_SC)

    tc_f = pl.pallas_call(
        _tc_manual,
        in_specs=[pl.BlockSpec(memory_space=pltpu.MemorySpace.HBM)],
        out_shape=jax.ShapeDtypeStruct(((R - R_SC) // 8, 1, 8), jnp.int32),
        scratch_shapes=[
            pltpu.VMEM((BR, N), jnp.float32),
            pltpu.VMEM((BR, N), jnp.float32),
            pltpu.VMEM((BR, N), jnp.float32),
            pltpu.VMEM((BR, N), jnp.float32),
            pltpu.SemaphoreType.DMA,
            pltpu.SemaphoreType.DMA,
            pltpu.SemaphoreType.DMA,
            pltpu.SemaphoreType.DMA,
        ],
    )
    idx_tc = tc_f(x2d).reshape(R - R_SC)
    return jnp.concatenate([idx_sc, idx_tc])


def kernel(x):
    idx = _argmax_split(x.reshape(R, N))
    return idx.reshape(128, 8).astype(jnp.int64)
